# baseline (speedup 1.0000x reference)
import jax
import jax.numpy as jnp
from jax.experimental import pallas as pl
from jax.experimental.pallas import tpu as pltpu


def _cell_kernel(Ace_ref, Aco_ref, Aee_ref, Aeo_ref, Aoe_ref, Aoo_ref,
                 xTe_ref, xTo_ref, hg_ref,
                 w1x_ref, W1h_ref, b1_ref,
                 w2x_ref, W2h_ref, b2_ref,
                 out_ref,
                 axf_s, axe_s, axo_s):
    b = pl.program_id(0)
    nb = pl.num_programs(0)
    f32 = jnp.float32
    bf16 = jnp.bfloat16

    @pl.when(b == 0)
    def _():
        axf_s[...] = (jnp.dot(Ace_ref[...], xTe_ref[...], preferred_element_type=f32)
                      + jnp.dot(Aco_ref[...], xTo_ref[...], preferred_element_type=f32))
        axe_s[...] = (jnp.dot(Aee_ref[...], xTe_ref[...], preferred_element_type=f32)
                      + jnp.dot(Aeo_ref[...], xTo_ref[...], preferred_element_type=f32))
        axo_s[...] = (jnp.dot(Aoe_ref[...], xTe_ref[...], preferred_element_type=f32)
                      + jnp.dot(Aoo_ref[...], xTo_ref[...], preferred_element_type=f32))

    onehot = (jax.lax.broadcasted_iota(jnp.int32, (nb, 1), 0) == b).astype(f32)
    axc = jnp.dot(axf_s[...], onehot, preferred_element_type=f32)
    axce = jnp.dot(axe_s[...], onehot, preferred_element_type=f32)
    axco = jnp.dot(axo_s[...], onehot, preferred_element_type=f32)

    hg = hg_ref[0]

    n_half, h2 = hg.shape
    hdim = h2 // 2

    h_lo = hg[:, :hdim].astype(bf16)
    h_hi = hg[:, hdim:].astype(bf16)

    ah = (jnp.dot(Ace_ref[...], h_lo, preferred_element_type=f32)
          + jnp.dot(Aco_ref[...], h_hi, preferred_element_type=f32))
    pre1 = jnp.dot(ah.astype(bf16), W1h_ref[...], preferred_element_type=f32)
    pre1 = pre1 + axc * w1x_ref[...] + b1_ref[...]
    s = jax.nn.sigmoid(pre1)

    rh = (s[:n_half, :] * hg).astype(bf16)
    u = s[n_half:, :]

    rh_lo = rh[:, :hdim]
    rh_hi = rh[:, hdim:]

    pe = (jnp.dot(Aee_ref[...], rh_lo, preferred_element_type=f32)
          + jnp.dot(Aeo_ref[...], rh_hi, preferred_element_type=f32))
    po = (jnp.dot(Aoe_ref[...], rh_lo, preferred_element_type=f32)
          + jnp.dot(Aoo_ref[...], rh_hi, preferred_element_type=f32))

    ce = jnp.tanh(jnp.dot(pe.astype(bf16), W2h_ref[...], preferred_element_type=f32)
                  + axce * w2x_ref[...] + b2_ref[...])
    co = jnp.tanh(jnp.dot(po.astype(bf16), W2h_ref[...], preferred_element_type=f32)
                  + axco * w2x_ref[...] + b2_ref[...])
    c = jnp.concatenate([ce, co], axis=1)

    out_ref[0] = u * hg + (1.0 - u) * c


@jax.jit
def kernel(inputs, hidden_state, view, W1, b1, W2, b2):
    B, N = inputs.shape
    H = W2.shape[1]
    Nh = N // 2
    H2 = 2 * H

    bf16 = jnp.bfloat16
    Ab = view.astype(bf16)
    hg = hidden_state.reshape(B, Nh, H2)
    xT = inputs.T.astype(bf16)
    xTe = xT[0::2]
    xTo = xT[1::2]
    Ace = Ab[:, 0::2]
    Aco = Ab[:, 1::2]
    Aee = Ab[0::2, 0::2]
    Aeo = Ab[0::2, 1::2]
    Aoe = Ab[1::2, 0::2]
    Aoo = Ab[1::2, 1::2]
    w1x = W1[0:1]
    W1h = W1[1:].astype(bf16)
    w2x = W2[0:1]
    W2h = W2[1:].astype(bf16)
    b1r = b1.reshape(1, H2)
    b2r = b2.reshape(1, H)

    def const(shape):
        nzeros = (0,) * len(shape)
        return pl.BlockSpec(shape, lambda b, _z=nzeros: _z)

    out = pl.pallas_call(
        _cell_kernel,
        grid=(B,),
        in_specs=[
            const((N, Nh)), const((N, Nh)),
            const((Nh, Nh)), const((Nh, Nh)), const((Nh, Nh)), const((Nh, Nh)),
            const((Nh, B)), const((Nh, B)),
            pl.BlockSpec((1, Nh, H2), lambda b: (b, 0, 0)),
            const((1, H2)), const((H, H2)), const((1, H2)),
            const((1, H)), const((H, H)), const((1, H)),
        ],
        out_specs=pl.BlockSpec((1, Nh, H2), lambda b: (b, 0, 0)),
        out_shape=jax.ShapeDtypeStruct((B, Nh, H2), jnp.float32),
        scratch_shapes=[
            pltpu.VMEM((N, B), jnp.float32),
            pltpu.VMEM((Nh, B), jnp.float32),
            pltpu.VMEM((Nh, B), jnp.float32),
        ],
    )(Ace, Aco, Aee, Aeo, Aoe, Aoo, xTe, xTo, hg,
      w1x, W1h, b1r, w2x, W2h, b2r)
    return out.reshape(B, N * H)


# speedup vs baseline: 1.4138x; 1.4138x over previous
import jax
import jax.numpy as jnp
from jax.experimental import pallas as pl
from jax.experimental.pallas import tpu as pltpu


def _cell_kernel(A_ref, App_ref, xTn_ref, xTP_ref, h3_ref, hg_ref,
                 w1x_ref, W1h_ref, b1_ref,
                 w2x_ref, W2h_ref, b2_ref,
                 out_ref,
                 axf_s, axP_s):
    b = pl.program_id(0)
    nb = pl.num_programs(0)
    f32 = jnp.float32
    bf16 = jnp.bfloat16

    @pl.when(b == 0)
    def _():
        axf_s[...] = jnp.dot(A_ref[...], xTn_ref[...], preferred_element_type=f32)
        axP_s[...] = jnp.dot(App_ref[...], xTP_ref[...], preferred_element_type=f32)

    onehot = (jax.lax.broadcasted_iota(jnp.int32, (nb, 1), 0) == b).astype(f32)
    axc = jnp.dot(axf_s[...], onehot, preferred_element_type=f32)
    axcP = jnp.dot(axP_s[...], onehot, preferred_element_type=f32)

    h3 = h3_ref[0]
    hg = hg_ref[0]

    n_half, h2 = hg.shape
    hdim = h2 // 2

    ah = jnp.dot(A_ref[...], h3.astype(bf16), preferred_element_type=f32)
    pre1 = jnp.dot(ah.astype(bf16), W1h_ref[...], preferred_element_type=f32)
    pre1 = pre1 + axc * w1x_ref[...] + b1_ref[...]
    s = jax.nn.sigmoid(pre1)

    rh = (s[:n_half, :] * hg).astype(bf16)
    u = s[n_half:, :]

    rh_P = jnp.concatenate([rh[:, :hdim], rh[:, hdim:]], axis=0)

    pP = jnp.dot(App_ref[...], rh_P, preferred_element_type=f32)
    cP = jnp.tanh(jnp.dot(pP.astype(bf16), W2h_ref[...], preferred_element_type=f32)
                  + axcP * w2x_ref[...] + b2_ref[...])
    c = jnp.concatenate([cP[:n_half], cP[n_half:]], axis=1)

    out_ref[0] = u * hg + (1.0 - u) * c


@jax.jit
def kernel(inputs, hidden_state, view, W1, b1, W2, b2):
    B, N = inputs.shape
    H = W2.shape[1]
    Nh = N // 2
    H2 = 2 * H

    bf16 = jnp.bfloat16
    perm = jnp.concatenate([jnp.arange(0, N, 2), jnp.arange(1, N, 2)])
    Ab = view.astype(bf16)
    App = Ab[perm][:, perm]
    h3 = hidden_state.reshape(B, N, H)
    hg = hidden_state.reshape(B, Nh, H2)
    xTn = inputs.T.astype(bf16)
    xTP = xTn[perm]
    w1x = W1[0:1]
    W1h = W1[1:].astype(bf16)
    w2x = W2[0:1]
    W2h = W2[1:].astype(bf16)
    b1r = b1.reshape(1, H2)
    b2r = b2.reshape(1, H)

    def const(shape):
        nzeros = (0,) * len(shape)
        return pl.BlockSpec(shape, lambda b, _z=nzeros: _z)

    out = pl.pallas_call(
        _cell_kernel,
        grid=(B,),
        in_specs=[
            const((N, N)), const((N, N)),
            const((N, B)), const((N, B)),
            pl.BlockSpec((1, N, H), lambda b: (b, 0, 0)),
            pl.BlockSpec((1, Nh, H2), lambda b: (b, 0, 0)),
            const((1, H2)), const((H, H2)), const((1, H2)),
            const((1, H)), const((H, H)), const((1, H)),
        ],
        out_specs=pl.BlockSpec((1, Nh, H2), lambda b: (b, 0, 0)),
        out_shape=jax.ShapeDtypeStruct((B, Nh, H2), jnp.float32),
        scratch_shapes=[
            pltpu.VMEM((N, B), jnp.float32),
            pltpu.VMEM((N, B), jnp.float32),
        ],
    )(Ab, App, xTn, xTP, h3, hg,
      w1x, W1h, b1r, w2x, W2h, b2r)
    return out.reshape(B, N * H)
